# Initial kernel scaffold; baseline (speedup 1.0000x reference)
import jax
import jax.numpy as jnp
from jax.experimental import pallas as pl


def kernel(prototypes, prototype_labels, query_feat, query_y):
    raise NotImplementedError("write your pallas kernel here")


# speedup vs baseline: 16.6801x; 16.6801x over previous
import functools

import jax
import jax.numpy as jnp
from jax.experimental import pallas as pl
from jax.experimental.pallas import tpu as pltpu

_N_CLASSES = 3
_K = 200
_SIGMA = 1.0
_ALPHA = 0.99
_FEAT = 192
_NPROTO = 300
_NPTS = 2048
_N = _NPROTO + _NPTS
_NPAD = 2432
_CPAD = 8
_N_ITERS = 96


def _tti_kernel(nf_ref, y_ref, qy_ref, pred_ref, loss_ref):
    nf = nf_ref[...]
    sq = jnp.sum(nf * nf, axis=1, keepdims=True)
    g = jax.lax.dot_general(nf, nf, (((1,), (1,)), ((), ())),
                            preferred_element_type=jnp.float32)
    d2 = sq + jnp.reshape(sq, (1, _NPAD)) - 2.0 * g
    d2 = jnp.maximum(d2, 0.0)

    rows = jax.lax.broadcasted_iota(jnp.int32, (_NPAD, _NPAD), 0)
    cols = jax.lax.broadcasted_iota(jnp.int32, (_NPAD, _NPAD), 1)
    invalid = (rows == cols) | (rows >= _N) | (cols >= _N)
    d2 = jnp.where(invalid, jnp.float32(1e30), d2)

    bits = jnp.maximum(jax.lax.bitcast_convert_type(d2, jnp.int32), 0)

    def bs_body(_, lohi):
        lo, hi = lohi
        mid = lo + ((hi - lo) >> 1)
        cnt = jnp.sum((bits <= mid).astype(jnp.int32), axis=1, keepdims=True)
        ge = cnt >= _K
        return jnp.where(ge, lo, mid + 1), jnp.where(ge, mid, hi)

    lo0 = jnp.zeros((_NPAD, 1), jnp.int32)
    hi0 = jnp.full((_NPAD, 1), 0x7F800000, jnp.int32)
    _, thr = jax.lax.fori_loop(0, 31, bs_body, (lo0, hi0))

    w = jnp.exp(d2 * (-0.5 / (_SIGMA * _SIGMA)))
    m_row = (bits <= thr).astype(jnp.float32)
    m_col = (bits <= jnp.reshape(thr, (1, _NPAD))).astype(jnp.float32)
    a_sym = w * (m_row + m_col)

    deg = jnp.sum(a_sym, axis=1, keepdims=True)
    s = jnp.sqrt(1.0 / (deg + 1e-8))
    smat = a_sym * s * jnp.reshape(s, (1, _NPAD))

    y = y_ref[...]
    theta = jnp.float32(1.0)
    delta = jnp.float32(_ALPHA)
    sigma1 = theta / delta

    z0 = jnp.zeros_like(y)
    r0 = y
    d0 = r0 / theta
    rho0 = 1.0 / sigma1

    def cheb_body(_, carry):
        z, r, d, rho = carry
        z = z + d
        sd = jax.lax.dot_general(smat, d, (((1,), (0,)), ((), ())),
                                 preferred_element_type=jnp.float32)
        r = r - (d - _ALPHA * sd)
        rho_new = 1.0 / (2.0 * sigma1 - rho)
        d = (rho_new * rho) * d + (2.0 * rho_new / delta) * r
        return z, r, d, rho_new

    z, _, _, _ = jax.lax.fori_loop(0, _N_ITERS, cheb_body, (z0, r0, d0, rho0))

    zq = z[0:_NPTS, :]
    pred_ref[...] = zq

    l0 = zq[:, 0:1]
    l1 = zq[:, 1:2]
    l2 = zq[:, 2:3]
    mx = jnp.maximum(l0, jnp.maximum(l1, l2))
    lse = mx + jnp.log(jnp.exp(l0 - mx) + jnp.exp(l1 - mx) + jnp.exp(l2 - mx))
    qy = qy_ref[...]
    chosen = jnp.where(qy == 0, l0, jnp.where(qy == 1, l1, l2))
    loss_ref[...] = jnp.sum(lse - chosen, axis=0, keepdims=True) * (1.0 / _NPTS)


@functools.partial(jax.jit, static_argnames=())
def kernel(prototypes, prototype_labels, query_feat, query_y):
    nf = jnp.concatenate([query_feat, prototypes], axis=0)
    nf = jnp.pad(nf, ((0, _NPAD - _N), (0, 0)))
    y = jnp.pad(prototype_labels,
                ((_NPTS, _NPAD - _N), (0, _CPAD - _N_CLASSES)))
    qy = jnp.reshape(query_y, (_NPTS, 1)).astype(jnp.int32)

    zq, loss = pl.pallas_call(
        _tti_kernel,
        out_shape=[
            jax.ShapeDtypeStruct((_NPTS, _CPAD), jnp.float32),
            jax.ShapeDtypeStruct((1, 1), jnp.float32),
        ],
        compiler_params=pltpu.CompilerParams(
            vmem_limit_bytes=100 * 1024 * 1024,
        ),
    )(nf, y, qy)

    pred = zq[:, :_N_CLASSES].reshape(1, _NPTS, _N_CLASSES).transpose(0, 2, 1)
    return (pred, loss[0, 0])


# speedup vs baseline: 60.7560x; 3.6424x over previous
import functools

import jax
import jax.numpy as jnp
from jax.experimental import pallas as pl
from jax.experimental.pallas import tpu as pltpu

_N_CLASSES = 3
_K = 200
_SIGMA = 1.0
_ALPHA = 0.99
_FEAT = 192
_NPROTO = 300
_NPTS = 2048
_N = _NPROTO + _NPTS
_NPAD = 2432
_CPAD = 8
_MAX_ITERS = 160
_RTOL2 = 2.5e-11


def _tti_kernel(nf_ref, y_ref, qy_ref, pred_ref, loss_ref):
    nf = nf_ref[...]
    sq = jnp.sum(nf * nf, axis=1, keepdims=True)
    g = jax.lax.dot_general(nf, nf, (((1,), (1,)), ((), ())),
                            preferred_element_type=jnp.float32)
    d2 = sq + jnp.reshape(sq, (1, _NPAD)) - 2.0 * g
    d2 = jnp.maximum(d2, 0.0)

    rows = jax.lax.broadcasted_iota(jnp.int32, (_NPAD, _NPAD), 0)
    cols = jax.lax.broadcasted_iota(jnp.int32, (_NPAD, _NPAD), 1)
    invalid = (rows == cols) | (rows >= _N) | (cols >= _N)
    d2 = jnp.where(invalid, jnp.float32(1e30), d2)

    bits = jnp.maximum(jax.lax.bitcast_convert_type(d2, jnp.int32), 0)

    def bs_body(_, lohi):
        lo, hi = lohi
        mid = lo + ((hi - lo) >> 1)
        cnt = jnp.sum((bits <= mid).astype(jnp.int32), axis=1, keepdims=True)
        ge = cnt >= _K
        return jnp.where(ge, lo, mid + 1), jnp.where(ge, mid, hi)

    lo0 = jnp.zeros((_NPAD, 1), jnp.int32)
    hi0 = jnp.full((_NPAD, 1), 0x7F800000, jnp.int32)
    _, thr = jax.lax.fori_loop(0, 31, bs_body, (lo0, hi0))

    w = jnp.exp(d2 * (-0.5 / (_SIGMA * _SIGMA)))
    m_row = (bits <= thr).astype(jnp.float32)
    m_col = (bits <= jnp.reshape(thr, (1, _NPAD))).astype(jnp.float32)
    a_sym = w * (m_row + m_col)

    deg = jnp.sum(a_sym, axis=1, keepdims=True)
    s = jnp.sqrt(1.0 / (deg + 1e-8))
    smat = a_sym * s * jnp.reshape(s, (1, _NPAD))

    y = y_ref[...]
    theta = jnp.float32(1.0)
    delta = jnp.float32(_ALPHA)
    sigma1 = theta / delta

    z0 = jnp.zeros_like(y)
    r0 = y
    d0 = r0 / theta
    rho0 = 1.0 / sigma1
    yy = jnp.sum(y * y)
    tol2 = _RTOL2 * yy

    def cheb_cond(carry):
        k, _, _, _, _, rr = carry
        return jnp.logical_and(k < _MAX_ITERS, rr > tol2)

    def cheb_body(carry):
        k, z, r, d, rho, _ = carry
        z = z + d
        sd = jax.lax.dot_general(smat, d, (((1,), (0,)), ((), ())),
                                 preferred_element_type=jnp.float32)
        r = r - (d - _ALPHA * sd)
        rho_new = 1.0 / (2.0 * sigma1 - rho)
        d = (rho_new * rho) * d + (2.0 * rho_new / delta) * r
        return k + 1, z, r, d, rho_new, jnp.sum(r * r)

    _, z, _, _, _, _ = jax.lax.while_loop(
        cheb_cond, cheb_body, (jnp.int32(0), z0, r0, d0, rho0, yy))

    zq = z[0:_NPTS, :]
    pred_ref[...] = zq

    l0 = zq[:, 0:1]
    l1 = zq[:, 1:2]
    l2 = zq[:, 2:3]
    mx = jnp.maximum(l0, jnp.maximum(l1, l2))
    lse = mx + jnp.log(jnp.exp(l0 - mx) + jnp.exp(l1 - mx) + jnp.exp(l2 - mx))
    qy = qy_ref[...]
    chosen = jnp.where(qy == 0, l0, jnp.where(qy == 1, l1, l2))
    loss_ref[...] = jnp.sum(lse - chosen, axis=0, keepdims=True) * (1.0 / _NPTS)


@functools.partial(jax.jit, static_argnames=())
def kernel(prototypes, prototype_labels, query_feat, query_y):
    nf = jnp.concatenate([query_feat, prototypes], axis=0)
    nf = jnp.pad(nf, ((0, _NPAD - _N), (0, 0)))
    y = jnp.pad(prototype_labels,
                ((_NPTS, _NPAD - _N), (0, _CPAD - _N_CLASSES)))
    qy = jnp.reshape(query_y, (_NPTS, 1)).astype(jnp.int32)

    zq, loss = pl.pallas_call(
        _tti_kernel,
        out_shape=[
            jax.ShapeDtypeStruct((_NPTS, _CPAD), jnp.float32),
            jax.ShapeDtypeStruct((1, 1), jnp.float32),
        ],
        compiler_params=pltpu.CompilerParams(
            vmem_limit_bytes=100 * 1024 * 1024,
        ),
    )(nf, y, qy)

    pred = zq[:, :_N_CLASSES].reshape(1, _NPTS, _N_CLASSES).transpose(0, 2, 1)
    return (pred, loss[0, 0])


# speedup vs baseline: 387.7224x; 6.3816x over previous
import functools

import jax
import jax.numpy as jnp
from jax.experimental import pallas as pl
from jax.experimental.pallas import tpu as pltpu

_N_CLASSES = 3
_K = 200
_SIGMA = 1.0
_ALPHA = 0.99
_FEAT = 192
_NPROTO = 300
_NPTS = 2048
_N = _NPROTO + _NPTS
_NPAD = 2432
_CPAD = 8
_MAX_ITERS = 160
_RTOL2 = 1e-12


def _tti_kernel(nf_ref, y_ref, qy_ref, pred_ref, loss_ref):
    nf = nf_ref[...]
    sq = jnp.sum(nf * nf, axis=1, keepdims=True)
    g = jax.lax.dot_general(nf, nf, (((1,), (1,)), ((), ())),
                            preferred_element_type=jnp.float32)
    d2 = sq + jnp.reshape(sq, (1, _NPAD)) - 2.0 * g
    d2 = jnp.maximum(d2, 0.0)

    rows = jax.lax.broadcasted_iota(jnp.int32, (_NPAD, _NPAD), 0)
    cols = jax.lax.broadcasted_iota(jnp.int32, (_NPAD, _NPAD), 1)
    invalid = (rows == cols) | (rows >= _N) | (cols >= _N)
    d2 = jnp.where(invalid, jnp.float32(1e30), d2)

    y = y_ref[...]

    gmin = jnp.min(d2)

    def _solve_full(_):
        bits = jnp.maximum(jax.lax.bitcast_convert_type(d2, jnp.int32), 0)

        def bs_body(_, lohi):
            lo, hi = lohi
            mid = lo + ((hi - lo) >> 1)
            cnt = jnp.sum((bits <= mid).astype(jnp.int32), axis=1,
                          keepdims=True)
            ge = cnt >= _K
            return jnp.where(ge, lo, mid + 1), jnp.where(ge, mid, hi)

        lo0 = jnp.zeros((_NPAD, 1), jnp.int32)
        hi0 = jnp.full((_NPAD, 1), 0x7F800000, jnp.int32)
        _, thr = jax.lax.fori_loop(0, 31, bs_body, (lo0, hi0))

        w = jnp.exp(d2 * (-0.5 / (_SIGMA * _SIGMA)))
        m_row = (bits <= thr).astype(jnp.float32)
        m_col = (bits <= jnp.reshape(thr, (1, _NPAD))).astype(jnp.float32)
        a_sym = w * (m_row + m_col)

        deg = jnp.sum(a_sym, axis=1, keepdims=True)
        s = jnp.sqrt(1.0 / (deg + 1e-8))
        smat = a_sym * s * jnp.reshape(s, (1, _NPAD))

        theta = jnp.float32(1.0)
        delta = jnp.float32(_ALPHA)
        sigma1 = theta / delta

        z0 = jnp.zeros_like(y)
        r0 = y
        d0 = r0 / theta
        rho0 = 1.0 / sigma1
        yy = jnp.sum(y * y)
        tol2 = _RTOL2 * yy

        def cheb_cond(carry):
            k, _, _, _, _, rr = carry
            return jnp.logical_and(k < _MAX_ITERS, rr > tol2)

        def cheb_body(carry):
            k, z, r, d, rho, _ = carry
            z = z + d
            sd = jax.lax.dot_general(smat, d, (((1,), (0,)), ((), ())),
                                     preferred_element_type=jnp.float32)
            r = r - (d - _ALPHA * sd)
            rho_new = 1.0 / (2.0 * sigma1 - rho)
            d = (rho_new * rho) * d + (2.0 * rho_new / delta) * r
            return k + 1, z, r, d, rho_new, jnp.sum(r * r)

        _, z, _, _, _, _ = jax.lax.while_loop(
            cheb_cond, cheb_body, (jnp.int32(0), z0, r0, d0, rho0, yy))
        return z

    def _solve_trivial(_):
        return y

    z = jax.lax.cond(gmin > 140.0, _solve_trivial, _solve_full, None)

    zq = z[0:_NPTS, :]
    pred_ref[...] = zq

    l0 = zq[:, 0:1]
    l1 = zq[:, 1:2]
    l2 = zq[:, 2:3]
    mx = jnp.maximum(l0, jnp.maximum(l1, l2))
    lse = mx + jnp.log(jnp.exp(l0 - mx) + jnp.exp(l1 - mx) + jnp.exp(l2 - mx))
    qy = qy_ref[...]
    chosen = jnp.where(qy == 0, l0, jnp.where(qy == 1, l1, l2))
    loss_ref[...] = jnp.sum(lse - chosen, axis=0, keepdims=True) * (1.0 / _NPTS)


@functools.partial(jax.jit, static_argnames=())
def kernel(prototypes, prototype_labels, query_feat, query_y):
    nf = jnp.concatenate([query_feat, prototypes], axis=0)
    nf = jnp.pad(nf, ((0, _NPAD - _N), (0, 0)))
    y = jnp.pad(prototype_labels,
                ((_NPTS, _NPAD - _N), (0, _CPAD - _N_CLASSES)))
    qy = jnp.reshape(query_y, (_NPTS, 1)).astype(jnp.int32)

    zq, loss = pl.pallas_call(
        _tti_kernel,
        out_shape=[
            jax.ShapeDtypeStruct((_NPTS, _CPAD), jnp.float32),
            jax.ShapeDtypeStruct((1, 1), jnp.float32),
        ],
        compiler_params=pltpu.CompilerParams(
            vmem_limit_bytes=100 * 1024 * 1024,
        ),
    )(nf, y, qy)

    pred = zq[:, :_N_CLASSES].reshape(1, _NPTS, _N_CLASSES).transpose(0, 2, 1)
    return (pred, loss[0, 0])
